# baseline (speedup 1.0000x reference)
import functools

import jax
import jax.numpy as jnp
from jax import lax
from jax.experimental import pallas as pl
from jax.experimental.pallas import tpu as pltpu
from jax.experimental.pallas import tpu_sc as plsc

N_ATOMS = 10000
N_BONDS = 320000
D = 128

NC = 2
NS = 16
NW = NC * NS

CH = 128
GROUPS = 80
ROWS_PER_W = CH * GROUPS
BONDS_PAD = ROWS_PER_W * NW
PAD = BONDS_PAD - N_BONDS
DUMMY = 64
ACC_ROWS = N_ATOMS + DUMMY

MLP_BLOCK = 1280
MLP_GRID_REAL = N_BONDS // MLP_BLOCK
MLP_GRID = BONDS_PAD // MLP_BLOCK

INIT_TILES = 10
INIT_ROWS = N_ATOMS // INIT_TILES
COMBINE_BLOCK = 1000


def _mlp_body(x_ref, wm_ref, bm_ref, wg_ref, bg_ref, o_ref):
    x = x_ref[...].astype(jnp.bfloat16)
    zm = jnp.dot(x, wm_ref[...].astype(jnp.bfloat16),
                 preferred_element_type=jnp.float32) + bm_ref[...]
    zg = jnp.dot(x, wg_ref[...].astype(jnp.bfloat16),
                 preferred_element_type=jnp.float32) + bg_ref[...]
    o_ref[...] = zm * jax.nn.sigmoid(zm) * jax.nn.sigmoid(zg)


def _mlp(bond_features, W_main, b_main, W_gate, b_gate):
    return pl.pallas_call(
        _mlp_body,
        grid=(MLP_GRID,),
        in_specs=[
            pl.BlockSpec((MLP_BLOCK, D),
                         lambda i: (jnp.minimum(i, MLP_GRID_REAL - 1), 0)),
            pl.BlockSpec((D, D), lambda i: (0, 0)),
            pl.BlockSpec((1, D), lambda i: (0, 0)),
            pl.BlockSpec((D, D), lambda i: (0, 0)),
            pl.BlockSpec((1, D), lambda i: (0, 0)),
        ],
        out_specs=pl.BlockSpec((MLP_BLOCK, D), lambda i: (i, 0)),
        out_shape=jax.ShapeDtypeStruct((BONDS_PAD, D), jnp.float32),
    )(bond_features, W_main, b_main.reshape(1, D), W_gate, b_gate.reshape(1, D))


def _sc_scatter_body(msg_hbm, dst_hbm, atom_hbm, out_hbm, acc_sh, idx_v, buf_v,
                     sem0, sem1):
    c = lax.axis_index("c")
    s = lax.axis_index("s")
    w = s * NC + c
    base = w * ROWS_PER_W
    @pl.when(s < INIT_TILES)
    def _init():
        pltpu.sync_copy(atom_hbm.at[pl.ds(s * INIT_ROWS, INIT_ROWS)],
                        acc_sh.at[pl.ds(s * INIT_ROWS, INIT_ROWS)])

    pltpu.sync_copy(dst_hbm.at[pl.ds(w * GROUPS, GROUPS)], idx_v)
    plsc.subcore_barrier()

    sems = (sem0, sem1)
    pltpu.async_copy(msg_hbm.at[pl.ds(base, CH)], buf_v.at[0], sems[0])

    def pair(k, carry):
        for b in range(2):
            g = 2 * k + b
            pltpu.make_async_copy(msg_hbm.at[pl.ds(base + g * CH, CH)],
                                  buf_v.at[b], sems[b]).wait()

            @pl.when(g + 1 < GROUPS)
            def _start_next():
                pltpu.async_copy(msg_hbm.at[pl.ds(base + (g + 1) * CH, CH)],
                                 buf_v.at[1 - b], sems[1 - b])

            pltpu.sync_copy(buf_v.at[b], acc_sh.at[idx_v.at[g]], add=True)
        return carry

    lax.fori_loop(0, GROUPS // 2, pair, 0)
    plsc.subcore_barrier()

    @pl.when(s < INIT_TILES)
    def _out():
        pltpu.sync_copy(acc_sh.at[pl.ds(s * INIT_ROWS, INIT_ROWS)],
                        out_hbm.at[c, pl.ds(s * INIT_ROWS, INIT_ROWS)])


_sc_scatter = functools.partial(
    pl.kernel,
    mesh=plsc.VectorSubcoreMesh(core_axis_name="c", subcore_axis_name="s"),
    out_type=jax.ShapeDtypeStruct((NC, N_ATOMS, D), jnp.float32),
    scratch_types=[
        pltpu.VMEM_SHARED((ACC_ROWS, D), jnp.float32),
        pltpu.VMEM((GROUPS, CH), jnp.int32),
        pltpu.VMEM((2, CH, D), jnp.float32),
        pltpu.SemaphoreType.DMA,
        pltpu.SemaphoreType.DMA,
    ],
)(_sc_scatter_body)


def _combine_body(p_ref, a_ref, o_ref):
    o_ref[...] = p_ref[0] + p_ref[1] - a_ref[...]


def _combine(partials, atom_features):
    return pl.pallas_call(
        _combine_body,
        grid=(N_ATOMS // COMBINE_BLOCK,),
        in_specs=[
            pl.BlockSpec((NC, COMBINE_BLOCK, D), lambda i: (0, i, 0)),
            pl.BlockSpec((COMBINE_BLOCK, D), lambda i: (i, 0)),
        ],
        out_specs=pl.BlockSpec((COMBINE_BLOCK, D), lambda i: (i, 0)),
        out_shape=jax.ShapeDtypeStruct((N_ATOMS, D), jnp.float32),
    )(partials, atom_features)


def kernel(atom_features, bond_features, bond_atom_indices, W_main, b_main, W_gate, b_gate):
    messages = _mlp(bond_features, W_main, b_main, W_gate, b_gate)
    dst = bond_atom_indices[:, 1]
    pad_idx = N_ATOMS + lax.rem(lax.iota(jnp.int32, PAD), jnp.int32(DUMMY))
    dst_pad = jnp.concatenate([dst, pad_idx]).reshape(BONDS_PAD // CH, CH)
    partials = _sc_scatter(messages, dst_pad, atom_features)
    return _combine(partials, atom_features)


# speedup vs baseline: 1.2100x; 1.2100x over previous
import functools

import jax
import jax.numpy as jnp
from jax import lax
from jax.experimental import pallas as pl
from jax.experimental.pallas import tpu as pltpu
from jax.experimental.pallas import tpu_sc as plsc

N_ATOMS = 10000
N_BONDS = 320000
D = 128

NC = 2
NS = 16
NW = NC * NS

CH = 128
GROUPS = 80
ROWS_PER_W = CH * GROUPS
BONDS_PAD = ROWS_PER_W * NW
PAD = BONDS_PAD - N_BONDS
DUMMY = 64
ACC_ROWS = N_ATOMS + DUMMY

MLP_BLOCK = 2560
MLP_GRID_REAL = N_BONDS // MLP_BLOCK
MLP_GRID = BONDS_PAD // MLP_BLOCK

INIT_TILES = 10
INIT_ROWS = N_ATOMS // INIT_TILES
COMBINE_BLOCK = 1000


def _mlp_body(x_ref, wm_ref, bm_ref, wg_ref, bg_ref, o_ref):
    x = x_ref[...].astype(jnp.bfloat16)
    zm = jnp.dot(x, wm_ref[...].astype(jnp.bfloat16),
                 preferred_element_type=jnp.float32) + bm_ref[...]
    zg = jnp.dot(x, wg_ref[...].astype(jnp.bfloat16),
                 preferred_element_type=jnp.float32) + bg_ref[...]
    o_ref[...] = zm * jax.nn.sigmoid(zm) * jax.nn.sigmoid(zg)


def _mlp(bond_features, W_main, b_main, W_gate, b_gate):
    return pl.pallas_call(
        _mlp_body,
        grid=(MLP_GRID,),
        in_specs=[
            pl.BlockSpec((MLP_BLOCK, D),
                         lambda i: (jnp.minimum(i, MLP_GRID_REAL - 1), 0)),
            pl.BlockSpec((D, D), lambda i: (0, 0)),
            pl.BlockSpec((1, D), lambda i: (0, 0)),
            pl.BlockSpec((D, D), lambda i: (0, 0)),
            pl.BlockSpec((1, D), lambda i: (0, 0)),
        ],
        out_specs=pl.BlockSpec((MLP_BLOCK, D), lambda i: (i, 0)),
        out_shape=jax.ShapeDtypeStruct((BONDS_PAD, D), jnp.float32),
    )(bond_features, W_main, b_main.reshape(1, D), W_gate, b_gate.reshape(1, D))


def _sc_scatter_body(msg_hbm, dst_hbm, atom_hbm, out_hbm, acc_sh, idx_v, buf_v,
                     sem0, sem1):
    c = lax.axis_index("c")
    s = lax.axis_index("s")
    w = s * NC + c
    base = w * ROWS_PER_W
    @pl.when(s < INIT_TILES)
    def _init():
        pltpu.sync_copy(atom_hbm.at[pl.ds(s * INIT_ROWS, INIT_ROWS)],
                        acc_sh.at[pl.ds(s * INIT_ROWS, INIT_ROWS)])

    pltpu.sync_copy(dst_hbm.at[pl.ds(w * GROUPS, GROUPS)], idx_v)
    plsc.subcore_barrier()

    sems = (sem0, sem1)
    pltpu.async_copy(msg_hbm.at[pl.ds(base, CH)], buf_v.at[0], sems[0])

    def pair(k, carry):
        for b in range(2):
            g = 2 * k + b
            pltpu.make_async_copy(msg_hbm.at[pl.ds(base + g * CH, CH)],
                                  buf_v.at[b], sems[b]).wait()

            @pl.when(g + 1 < GROUPS)
            def _start_next():
                pltpu.async_copy(msg_hbm.at[pl.ds(base + (g + 1) * CH, CH)],
                                 buf_v.at[1 - b], sems[1 - b])

            pltpu.sync_copy(buf_v.at[b], acc_sh.at[idx_v.at[g]], add=True)
        return carry

    lax.fori_loop(0, GROUPS // 2, pair, 0)
    plsc.subcore_barrier()

    @pl.when(s < INIT_TILES)
    def _out():
        pltpu.sync_copy(acc_sh.at[pl.ds(s * INIT_ROWS, INIT_ROWS)],
                        out_hbm.at[c, pl.ds(s * INIT_ROWS, INIT_ROWS)])


_sc_scatter = functools.partial(
    pl.kernel,
    mesh=plsc.VectorSubcoreMesh(core_axis_name="c", subcore_axis_name="s"),
    out_type=jax.ShapeDtypeStruct((NC, N_ATOMS, D), jnp.float32),
    scratch_types=[
        pltpu.VMEM_SHARED((ACC_ROWS, D), jnp.float32),
        pltpu.VMEM((GROUPS, CH), jnp.int32),
        pltpu.VMEM((2, CH, D), jnp.float32),
        pltpu.SemaphoreType.DMA,
        pltpu.SemaphoreType.DMA,
    ],
)(_sc_scatter_body)


def _combine_body(p_ref, a_ref, o_ref):
    o_ref[...] = p_ref[0] + p_ref[1] - a_ref[...]


def _combine(partials, atom_features):
    return pl.pallas_call(
        _combine_body,
        grid=(N_ATOMS // COMBINE_BLOCK,),
        in_specs=[
            pl.BlockSpec((NC, COMBINE_BLOCK, D), lambda i: (0, i, 0)),
            pl.BlockSpec((COMBINE_BLOCK, D), lambda i: (i, 0)),
        ],
        out_specs=pl.BlockSpec((COMBINE_BLOCK, D), lambda i: (i, 0)),
        out_shape=jax.ShapeDtypeStruct((N_ATOMS, D), jnp.float32),
    )(partials, atom_features)


def kernel(atom_features, bond_features, bond_atom_indices, W_main, b_main, W_gate, b_gate):
    messages = _mlp(bond_features, W_main, b_main, W_gate, b_gate)
    dst = bond_atom_indices[:, 1]
    pad_idx = N_ATOMS + lax.rem(lax.iota(jnp.int32, PAD), jnp.int32(DUMMY))
    dst_pad = jnp.concatenate([dst, pad_idx]).reshape(BONDS_PAD // CH, CH)
    partials = _sc_scatter(messages, dst_pad, atom_features)
    return _combine(partials, atom_features)


# speedup vs baseline: 1.4364x; 1.1871x over previous
import functools

import jax
import jax.numpy as jnp
from jax import lax
from jax.experimental import pallas as pl
from jax.experimental.pallas import tpu as pltpu
from jax.experimental.pallas import tpu_sc as plsc

N_ATOMS = 10000
N_BONDS = 320000
D = 128

NC = 2
NS = 16
NW = NC * NS

CH = 128
GROUPS = 80
ROWS_PER_W = CH * GROUPS
BONDS_PAD = ROWS_PER_W * NW
PAD = BONDS_PAD - N_BONDS
DUMMY = 64
ACC_ROWS = N_ATOMS + DUMMY

MLP_BLOCK = 8192
MLP_GRID = BONDS_PAD // MLP_BLOCK

INIT_TILES = 10
INIT_ROWS = N_ATOMS // INIT_TILES
COMBINE_BLOCK = 1000


def _mlp_body(x_ref, wm_ref, bm_ref, wg_ref, bg_ref, o_ref):
    x = x_ref[...].astype(jnp.bfloat16)
    zm = jnp.dot(x, wm_ref[...].astype(jnp.bfloat16),
                 preferred_element_type=jnp.float32) + bm_ref[...]
    zg = jnp.dot(x, wg_ref[...].astype(jnp.bfloat16),
                 preferred_element_type=jnp.float32) + bg_ref[...]
    o_ref[...] = zm * jax.nn.sigmoid(zm) * jax.nn.sigmoid(zg)


def _mlp(bond_features, W_main, b_main, W_gate, b_gate):
    return pl.pallas_call(
        _mlp_body,
        grid=(MLP_GRID,),
        in_specs=[
            pl.BlockSpec((MLP_BLOCK, D), lambda i: (i, 0)),
            pl.BlockSpec((D, D), lambda i: (0, 0)),
            pl.BlockSpec((1, D), lambda i: (0, 0)),
            pl.BlockSpec((D, D), lambda i: (0, 0)),
            pl.BlockSpec((1, D), lambda i: (0, 0)),
        ],
        out_specs=pl.BlockSpec((MLP_BLOCK, D), lambda i: (i, 0)),
        out_shape=jax.ShapeDtypeStruct((BONDS_PAD, D), jnp.float32),
    )(bond_features, W_main, b_main.reshape(1, D), W_gate, b_gate.reshape(1, D))


def _sc_scatter_body(msg_hbm, dst_hbm, atom_hbm, out_hbm, acc_sh, idx_v, buf_v,
                     sem0, sem1):
    c = lax.axis_index("c")
    s = lax.axis_index("s")
    w = s * NC + c
    base = w * ROWS_PER_W
    @pl.when(s < INIT_TILES)
    def _init():
        pltpu.sync_copy(atom_hbm.at[pl.ds(s * INIT_ROWS, INIT_ROWS)],
                        acc_sh.at[pl.ds(s * INIT_ROWS, INIT_ROWS)])

    pltpu.sync_copy(dst_hbm.at[pl.ds(w * GROUPS, GROUPS)], idx_v)
    plsc.subcore_barrier()

    sems = (sem0, sem1)
    pltpu.async_copy(msg_hbm.at[pl.ds(base, CH)], buf_v.at[0], sems[0])

    def pair(k, carry):
        for b in range(2):
            g = 2 * k + b
            pltpu.make_async_copy(msg_hbm.at[pl.ds(base + g * CH, CH)],
                                  buf_v.at[b], sems[b]).wait()

            @pl.when(g + 1 < GROUPS)
            def _start_next():
                pltpu.async_copy(msg_hbm.at[pl.ds(base + (g + 1) * CH, CH)],
                                 buf_v.at[1 - b], sems[1 - b])

            pltpu.sync_copy(buf_v.at[b], acc_sh.at[idx_v.at[g]], add=True)
        return carry

    lax.fori_loop(0, GROUPS // 2, pair, 0)
    plsc.subcore_barrier()

    @pl.when(s < INIT_TILES)
    def _out():
        pltpu.sync_copy(acc_sh.at[pl.ds(s * INIT_ROWS, INIT_ROWS)],
                        out_hbm.at[c, pl.ds(s * INIT_ROWS, INIT_ROWS)])


_sc_scatter = functools.partial(
    pl.kernel,
    mesh=plsc.VectorSubcoreMesh(core_axis_name="c", subcore_axis_name="s"),
    out_type=jax.ShapeDtypeStruct((NC, N_ATOMS, D), jnp.float32),
    scratch_types=[
        pltpu.VMEM_SHARED((ACC_ROWS, D), jnp.float32),
        pltpu.VMEM((GROUPS, CH), jnp.int32),
        pltpu.VMEM((2, CH, D), jnp.float32),
        pltpu.SemaphoreType.DMA,
        pltpu.SemaphoreType.DMA,
    ],
)(_sc_scatter_body)


def _combine_body(p_ref, a_ref, o_ref):
    o_ref[...] = p_ref[0] + p_ref[1] - a_ref[...]


def _combine(partials, atom_features):
    return pl.pallas_call(
        _combine_body,
        grid=(N_ATOMS // COMBINE_BLOCK,),
        in_specs=[
            pl.BlockSpec((NC, COMBINE_BLOCK, D), lambda i: (0, i, 0)),
            pl.BlockSpec((COMBINE_BLOCK, D), lambda i: (i, 0)),
        ],
        out_specs=pl.BlockSpec((COMBINE_BLOCK, D), lambda i: (i, 0)),
        out_shape=jax.ShapeDtypeStruct((N_ATOMS, D), jnp.float32),
    )(partials, atom_features)


def kernel(atom_features, bond_features, bond_atom_indices, W_main, b_main, W_gate, b_gate):
    messages = _mlp(bond_features, W_main, b_main, W_gate, b_gate)
    dst = bond_atom_indices[:, 1]
    pad_idx = N_ATOMS + lax.rem(lax.iota(jnp.int32, PAD), jnp.int32(DUMMY))
    dst_pad = jnp.concatenate([dst, pad_idx]).reshape(BONDS_PAD // CH, CH)
    partials = _sc_scatter(messages, dst_pad, atom_features)
    return _combine(partials, atom_features)


# speedup vs baseline: 1.4986x; 1.0433x over previous
import functools

import jax
import jax.numpy as jnp
from jax import lax
from jax.experimental import pallas as pl
from jax.experimental.pallas import tpu as pltpu
from jax.experimental.pallas import tpu_sc as plsc

N_ATOMS = 10000
N_BONDS = 320000
D = 128

NC = 2
NS = 16
NW = NC * NS

CH = 128
GROUPS = 80
ROWS_PER_W = CH * GROUPS
BONDS_PAD = ROWS_PER_W * NW
PAD = BONDS_PAD - N_BONDS
DUMMY = 64
ACC_ROWS = N_ATOMS + DUMMY

MLP_BLOCK = 16384
MLP_GRID = BONDS_PAD // MLP_BLOCK

INIT_TILES = 10
INIT_ROWS = N_ATOMS // INIT_TILES
COMBINE_BLOCK = 1000


def _mlp_body(x_ref, wm_ref, bm_ref, wg_ref, bg_ref, o_ref):
    x = x_ref[...].astype(jnp.bfloat16)
    zm = jnp.dot(x, wm_ref[...].astype(jnp.bfloat16),
                 preferred_element_type=jnp.float32) + bm_ref[...]
    zg = jnp.dot(x, wg_ref[...].astype(jnp.bfloat16),
                 preferred_element_type=jnp.float32) + bg_ref[...]
    o_ref[...] = zm * jax.nn.sigmoid(zm) * jax.nn.sigmoid(zg)


def _mlp(bond_features, W_main, b_main, W_gate, b_gate):
    return pl.pallas_call(
        _mlp_body,
        grid=(MLP_GRID,),
        in_specs=[
            pl.BlockSpec((MLP_BLOCK, D), lambda i: (i, 0)),
            pl.BlockSpec((D, D), lambda i: (0, 0)),
            pl.BlockSpec((1, D), lambda i: (0, 0)),
            pl.BlockSpec((D, D), lambda i: (0, 0)),
            pl.BlockSpec((1, D), lambda i: (0, 0)),
        ],
        out_specs=pl.BlockSpec((MLP_BLOCK, D), lambda i: (i, 0)),
        out_shape=jax.ShapeDtypeStruct((BONDS_PAD, D), jnp.float32),
    )(bond_features, W_main, b_main.reshape(1, D), W_gate, b_gate.reshape(1, D))


def _sc_scatter_body(msg_hbm, dst_hbm, atom_hbm, out_hbm, acc_sh, idx_v, buf_v,
                     sem0, sem1):
    c = lax.axis_index("c")
    s = lax.axis_index("s")
    w = s * NC + c
    base = w * ROWS_PER_W
    @pl.when(s < INIT_TILES)
    def _init():
        pltpu.sync_copy(atom_hbm.at[pl.ds(s * INIT_ROWS, INIT_ROWS)],
                        acc_sh.at[pl.ds(s * INIT_ROWS, INIT_ROWS)])

    pltpu.sync_copy(dst_hbm.at[pl.ds(w * GROUPS, GROUPS)], idx_v)
    plsc.subcore_barrier()

    sems = (sem0, sem1)
    pltpu.async_copy(msg_hbm.at[pl.ds(base, CH)], buf_v.at[0], sems[0])

    def pair(k, carry):
        for b in range(2):
            g = 2 * k + b
            pltpu.make_async_copy(msg_hbm.at[pl.ds(base + g * CH, CH)],
                                  buf_v.at[b], sems[b]).wait()

            @pl.when(g + 1 < GROUPS)
            def _start_next():
                pltpu.async_copy(msg_hbm.at[pl.ds(base + (g + 1) * CH, CH)],
                                 buf_v.at[1 - b], sems[1 - b])

            pltpu.sync_copy(buf_v.at[b], acc_sh.at[idx_v.at[g]], add=True)
        return carry

    lax.fori_loop(0, GROUPS // 2, pair, 0)
    plsc.subcore_barrier()

    @pl.when(s < INIT_TILES)
    def _out():
        pltpu.sync_copy(acc_sh.at[pl.ds(s * INIT_ROWS, INIT_ROWS)],
                        out_hbm.at[c, pl.ds(s * INIT_ROWS, INIT_ROWS)])


_sc_scatter = functools.partial(
    pl.kernel,
    mesh=plsc.VectorSubcoreMesh(core_axis_name="c", subcore_axis_name="s"),
    out_type=jax.ShapeDtypeStruct((NC, N_ATOMS, D), jnp.float32),
    scratch_types=[
        pltpu.VMEM_SHARED((ACC_ROWS, D), jnp.float32),
        pltpu.VMEM((GROUPS, CH), jnp.int32),
        pltpu.VMEM((2, CH, D), jnp.float32),
        pltpu.SemaphoreType.DMA,
        pltpu.SemaphoreType.DMA,
    ],
)(_sc_scatter_body)


def _combine_body(p_ref, a_ref, o_ref):
    o_ref[...] = p_ref[0] + p_ref[1] - a_ref[...]


def _combine(partials, atom_features):
    return pl.pallas_call(
        _combine_body,
        grid=(N_ATOMS // COMBINE_BLOCK,),
        in_specs=[
            pl.BlockSpec((NC, COMBINE_BLOCK, D), lambda i: (0, i, 0)),
            pl.BlockSpec((COMBINE_BLOCK, D), lambda i: (i, 0)),
        ],
        out_specs=pl.BlockSpec((COMBINE_BLOCK, D), lambda i: (i, 0)),
        out_shape=jax.ShapeDtypeStruct((N_ATOMS, D), jnp.float32),
    )(partials, atom_features)


def kernel(atom_features, bond_features, bond_atom_indices, W_main, b_main, W_gate, b_gate):
    messages = _mlp(bond_features, W_main, b_main, W_gate, b_gate)
    dst = bond_atom_indices[:, 1]
    pad_idx = N_ATOMS + lax.rem(lax.iota(jnp.int32, PAD), jnp.int32(DUMMY))
    dst_pad = jnp.concatenate([dst, pad_idx]).reshape(BONDS_PAD // CH, CH)
    partials = _sc_scatter(messages, dst_pad, atom_features)
    return _combine(partials, atom_features)


# speedup vs baseline: 1.5168x; 1.0121x over previous
import functools

import jax
import jax.numpy as jnp
import numpy as np
from jax import lax
from jax.experimental import pallas as pl
from jax.experimental.pallas import tpu as pltpu
from jax.experimental.pallas import tpu_sc as plsc

N_ATOMS = 10000
N_BONDS = 320000
D = 128

NC = 2
NS = 16
NW = NC * NS

CH = 128
N_CHUNKS = 2
GROUPS = 40
ROWS_PER_W = CH * GROUPS
CHUNK_ROWS = ROWS_PER_W * NW
BONDS_PAD = CHUNK_ROWS * N_CHUNKS
PAD = BONDS_PAD - N_BONDS
DUMMY = 64
ACC_ROWS = N_ATOMS + DUMMY

MLP_BLOCK = 16384
MLP_GRID = CHUNK_ROWS // MLP_BLOCK

INIT_TILES = 10
INIT_ROWS = N_ATOMS // INIT_TILES
COMBINE_BLOCK = 1000

_PAD_IDX = np.int32(N_ATOMS) + np.arange(PAD, dtype=np.int32) % np.int32(DUMMY)


def _mlp_body(x_ref, wm_ref, bm_ref, wg_ref, bg_ref, o_ref):
    x = x_ref[...].astype(jnp.bfloat16)
    zm = jnp.dot(x, wm_ref[...].astype(jnp.bfloat16),
                 preferred_element_type=jnp.float32) + bm_ref[...]
    zg = jnp.dot(x, wg_ref[...].astype(jnp.bfloat16),
                 preferred_element_type=jnp.float32) + bg_ref[...]
    o_ref[...] = zm * jax.nn.sigmoid(zm) * jax.nn.sigmoid(zg)


def _mlp_chunk(q, bond_features, W_main, b_main, W_gate, b_gate):
    return pl.pallas_call(
        _mlp_body,
        grid=(MLP_GRID,),
        in_specs=[
            pl.BlockSpec((MLP_BLOCK, D), lambda i: (i + q * MLP_GRID, 0)),
            pl.BlockSpec((D, D), lambda i: (0, 0)),
            pl.BlockSpec((1, D), lambda i: (0, 0)),
            pl.BlockSpec((D, D), lambda i: (0, 0)),
            pl.BlockSpec((1, D), lambda i: (0, 0)),
        ],
        out_specs=pl.BlockSpec((MLP_BLOCK, D), lambda i: (i, 0)),
        out_shape=jax.ShapeDtypeStruct((CHUNK_ROWS, D), jnp.float32),
    )(bond_features, W_main, b_main.reshape(1, D), W_gate, b_gate.reshape(1, D))


def _make_sc_scatter(q):
    def body(msg_hbm, dst_hbm, init_hbm, out_hbm, acc_sh, idx_v, buf_v,
             sem0, sem1):
        c = lax.axis_index("c")
        s = lax.axis_index("s")
        w = s * NC + c
        base = w * ROWS_PER_W

        @pl.when(s < INIT_TILES)
        def _init():
            if q == 0:
                src = init_hbm.at[pl.ds(s * INIT_ROWS, INIT_ROWS)]
            else:
                src = init_hbm.at[c, pl.ds(s * INIT_ROWS, INIT_ROWS)]
            pltpu.sync_copy(src, acc_sh.at[pl.ds(s * INIT_ROWS, INIT_ROWS)])

        pltpu.sync_copy(dst_hbm.at[pl.ds(q * NW * GROUPS + w * GROUPS, GROUPS)],
                        idx_v)
        plsc.subcore_barrier()

        sems = (sem0, sem1)
        pltpu.async_copy(msg_hbm.at[pl.ds(base, CH)], buf_v.at[0], sems[0])

        def pair(k, carry):
            for b in range(2):
                g = 2 * k + b
                pltpu.make_async_copy(msg_hbm.at[pl.ds(base + g * CH, CH)],
                                      buf_v.at[b], sems[b]).wait()

                @pl.when(g + 1 < GROUPS)
                def _start_next():
                    pltpu.async_copy(msg_hbm.at[pl.ds(base + (g + 1) * CH, CH)],
                                     buf_v.at[1 - b], sems[1 - b])

                pltpu.sync_copy(buf_v.at[b], acc_sh.at[idx_v.at[g]], add=True)
            return carry

        lax.fori_loop(0, GROUPS // 2, pair, 0)
        plsc.subcore_barrier()

        @pl.when(s < INIT_TILES)
        def _out():
            pltpu.sync_copy(acc_sh.at[pl.ds(s * INIT_ROWS, INIT_ROWS)],
                            out_hbm.at[c, pl.ds(s * INIT_ROWS, INIT_ROWS)])

    return pl.kernel(
        body,
        mesh=plsc.VectorSubcoreMesh(core_axis_name="c", subcore_axis_name="s"),
        out_type=jax.ShapeDtypeStruct((NC, N_ATOMS, D), jnp.float32),
        scratch_types=[
            pltpu.VMEM_SHARED((ACC_ROWS, D), jnp.float32),
            pltpu.VMEM((GROUPS, CH), jnp.int32),
            pltpu.VMEM((2, CH, D), jnp.float32),
            pltpu.SemaphoreType.DMA,
            pltpu.SemaphoreType.DMA,
        ],
    )


def _combine_body(p_ref, a_ref, o_ref):
    o_ref[...] = p_ref[0] + p_ref[1] - a_ref[...]


def _combine(partials, atom_features):
    return pl.pallas_call(
        _combine_body,
        grid=(N_ATOMS // COMBINE_BLOCK,),
        in_specs=[
            pl.BlockSpec((NC, COMBINE_BLOCK, D), lambda i: (0, i, 0)),
            pl.BlockSpec((COMBINE_BLOCK, D), lambda i: (i, 0)),
        ],
        out_specs=pl.BlockSpec((COMBINE_BLOCK, D), lambda i: (i, 0)),
        out_shape=jax.ShapeDtypeStruct((N_ATOMS, D), jnp.float32),
    )(partials, atom_features)


def kernel(atom_features, bond_features, bond_atom_indices, W_main, b_main, W_gate, b_gate):
    dst = bond_atom_indices[:, 1]
    dst_pad = jnp.concatenate([dst, jnp.asarray(_PAD_IDX)]).reshape(
        BONDS_PAD // CH, CH)
    msg0 = _mlp_chunk(0, bond_features, W_main, b_main, W_gate, b_gate)
    msg1 = _mlp_chunk(1, bond_features, W_main, b_main, W_gate, b_gate)
    p0 = _make_sc_scatter(0)(msg0, dst_pad, atom_features)
    p1 = _make_sc_scatter(1)(msg1, dst_pad, p0)
    return _combine(p1, atom_features)
